# Initial kernel scaffold; baseline (speedup 1.0000x reference)
import jax
import jax.numpy as jnp
from jax.experimental import pallas as pl


def kernel(edge_index, feat, coordinate, params):
    raise NotImplementedError("write your pallas kernel here")


# speedup vs baseline: 1.8966x; 1.8966x over previous
import dataclasses
import functools

import jax
import jax.numpy as jnp
from jax import lax
from jax.experimental import pallas as pl
from jax.experimental.pallas import tpu as pltpu
from jax.experimental.pallas import tpu_sc as plsc

N = 10000
E = 320000
H = 128
XP = 4
XF = N * XP
CH = 128
NCHUNK = E // CH
NC = 2
NS = 16
NW = NC * NS
ZRW = 632

_mesh = plsc.VectorSubcoreMesh(core_axis_name="c", subcore_axis_name="s")

_sc_params = pltpu.CompilerParams()
if "needs_layout_passes" in pltpu.CompilerParams.__dataclass_fields__:
    _sc_params = dataclasses.replace(_sc_params, needs_layout_passes=False)


def _silu(x):
    return x * (1.0 / (1.0 + jnp.exp(-x)))


def _sc_gather(h, xflat, src, dst):
    @functools.partial(
        pl.kernel,
        mesh=_mesh,
        compiler_params=_sc_params,
        out_type=(
            jax.ShapeDtypeStruct((E, H), jnp.float32),
            jax.ShapeDtypeStruct((E, H), jnp.float32),
            jax.ShapeDtypeStruct((E * XP,), jnp.float32),
            jax.ShapeDtypeStruct((E,), jnp.float32),
        ),
        scratch_types=[
            pltpu.VMEM((CH,), jnp.int32),
            pltpu.VMEM((CH,), jnp.int32),
            pltpu.VMEM((CH, H), jnp.float32),
            pltpu.VMEM((CH, H), jnp.float32),
            pltpu.VMEM((XF,), jnp.float32),
            pltpu.VMEM((CH * XP,), jnp.float32),
            pltpu.VMEM((CH,), jnp.float32),
            pltpu.SemaphoreType.DMA,
            pltpu.SemaphoreType.DMA,
        ],
    )
    def k(h_hbm, x_hbm, src_hbm, dst_hbm, hs_hbm, hd_hbm, df_hbm, r2_hbm,
          sidx, didx, hsr, hdr, xtab, dfb, r2b, s0, s1):
        wid = lax.axis_index("s") * NC + lax.axis_index("c")
        iota = lax.iota(jnp.int32, 16)

        pltpu.sync_copy(x_hbm, xtab)

        nloop = (NCHUNK + NW - 1) // NW

        @pl.loop(0, nloop)
        def _(t):
            c = wid + t * NW

            @pl.when(c < NCHUNK)
            def _():
                base = c * CH
                pltpu.sync_copy(src_hbm.at[pl.ds(base, CH)], sidx)
                pltpu.sync_copy(dst_hbm.at[pl.ds(base, CH)], didx)
                cp0 = pltpu.async_copy(h_hbm.at[sidx], hsr, s0)
                cp1 = pltpu.async_copy(h_hbm.at[didx], hdr, s1)

                @pl.loop(0, CH // 16)
                def _(g):
                    flat = (g * 16 + iota) * XP
                    srcv = sidx[pl.ds(g * 16, 16)] * XP
                    dstv = didx[pl.ds(g * 16, 16)] * XP
                    acc = jnp.zeros((16,), jnp.float32)
                    for cc in range(3):
                        colv = jnp.full((16,), cc, jnp.int32)
                        xsv = plsc.load_gather(xtab, [srcv + colv])
                        xdv = plsc.load_gather(xtab, [dstv + colv])
                        d = xdv - xsv
                        plsc.store_scatter(dfb, [flat + colv], d)
                        acc = acc + d * d
                    r2b[pl.ds(g * 16, 16)] = acc

                cp0.wait()
                cp1.wait()
                pltpu.sync_copy(hsr, hs_hbm.at[pl.ds(base, CH)])
                pltpu.sync_copy(hdr, hd_hbm.at[pl.ds(base, CH)])
                pltpu.sync_copy(dfb, df_hbm.at[pl.ds(base * XP, CH * XP)])
                pltpu.sync_copy(r2b, r2_hbm.at[pl.ds(base, CH)])

    return k(h, xflat, src, dst)


def _sc_scatter_m(m, dst, zrows):
    @functools.partial(
        pl.kernel,
        mesh=_mesh,
        compiler_params=_sc_params,
        out_type=jax.ShapeDtypeStruct((NC, N, H), jnp.float32),
        scratch_types=[
            pltpu.VMEM((CH,), jnp.int32),
            pltpu.VMEM((CH, H), jnp.float32),
            pltpu.VMEM_SHARED((N, H), jnp.float32),
        ],
    )
    def k(m_hbm, dst_hbm, z_hbm, magg_hbm, didx, mb, macc):
        cid = lax.axis_index("c")
        sid = lax.axis_index("s")

        zstart = pl.multiple_of(jnp.minimum(sid * ZRW, N - ZRW), 8)
        pltpu.sync_copy(z_hbm, macc.at[pl.ds(zstart, ZRW)])
        plsc.subcore_barrier()

        half = NCHUNK // NC
        nloop = (half + NS - 1) // NS

        @pl.loop(0, nloop)
        def _(t):
            c = cid * half + sid + t * NS

            @pl.when(c < (cid + 1) * half)
            def _():
                base = c * CH
                pltpu.sync_copy(dst_hbm.at[pl.ds(base, CH)], didx)
                pltpu.sync_copy(m_hbm.at[pl.ds(base, CH)], mb)
                pltpu.sync_copy(mb, macc.at[didx], add=True)

        plsc.subcore_barrier()
        pltpu.sync_copy(macc.at[pl.ds(zstart, ZRW)],
                        magg_hbm.at[cid].at[pl.ds(zstart, ZRW)])

    return k(m, dst, zrows)


def _sc_scatter_x(diff, w, dst, z1d):
    @functools.partial(
        pl.kernel,
        mesh=_mesh,
        compiler_params=_sc_params,
        out_type=jax.ShapeDtypeStruct((NW * XF,), jnp.float32),
        scratch_types=[
            pltpu.VMEM((CH,), jnp.int32),
            pltpu.VMEM((CH * XP,), jnp.float32),
            pltpu.VMEM((CH,), jnp.float32),
            pltpu.VMEM((XF,), jnp.float32),
        ],
    )
    def k(df_hbm, w_hbm, dst_hbm, z1_hbm, xacc_hbm, didx, dfb, wb, xacc_t):
        cid = lax.axis_index("c")
        sid = lax.axis_index("s")
        wid = sid * NC + cid
        iota = lax.iota(jnp.int32, 16)
        ones = jnp.ones((16,), jnp.float32)
        col3 = jnp.full((16,), 3, jnp.int32)

        pltpu.sync_copy(z1_hbm, xacc_t)

        nloop = (NCHUNK + NW - 1) // NW

        @pl.loop(0, nloop)
        def _(t):
            c = wid + t * NW

            @pl.when(c < NCHUNK)
            def _():
                base = c * CH
                pltpu.sync_copy(dst_hbm.at[pl.ds(base, CH)], didx)
                pltpu.sync_copy(df_hbm.at[pl.ds(base * XP, CH * XP)], dfb)
                pltpu.sync_copy(w_hbm.at[pl.ds(base, CH)], wb)

                @pl.loop(0, CH // 16)
                def _(g):
                    flat = (g * 16 + iota) * XP
                    dstv = didx[pl.ds(g * 16, 16)] * XP
                    wv = wb[pl.ds(g * 16, 16)]
                    for cc in range(3):
                        colv = jnp.full((16,), cc, jnp.int32)
                        dv = plsc.load_gather(dfb, [flat + colv])
                        plsc.addupdate_scatter(xacc_t, [dstv + colv], dv * wv)
                    plsc.addupdate_scatter(xacc_t, [dstv + col3], ones)

        pltpu.sync_copy(xacc_t, xacc_hbm.at[pl.ds(wid * XF, XF)])

    return k(diff, w, dst, z1d)


def _sc_xupdate(xflat, xacc_p):
    SPAN = 1264

    @functools.partial(
        pl.kernel,
        mesh=_mesh,
        compiler_params=_sc_params,
        out_type=jax.ShapeDtypeStruct((XF,), jnp.float32),
        scratch_types=[
            pltpu.VMEM((SPAN,), jnp.float32),
            pltpu.VMEM((SPAN,), jnp.float32),
            pltpu.VMEM((SPAN,), jnp.float32),
        ],
    )
    def k(x_hbm, acc_hbm, xo_hbm, xb, ab, sb):
        wid = lax.axis_index("s") * NC + lax.axis_index("c")
        start = jnp.minimum(wid * SPAN, XF - SPAN)
        iota = lax.iota(jnp.int32, 16)
        lane = iota % XP
        pltpu.sync_copy(x_hbm.at[pl.ds(start, SPAN)], xb)

        @pl.loop(0, SPAN // 16)
        def _(j):
            sb[pl.ds(j * 16, 16)] = jnp.zeros((16,), jnp.float32)

        @pl.loop(0, NW)
        def _(p):
            pltpu.sync_copy(acc_hbm.at[pl.ds(p * XF + start, SPAN)], ab)

            @pl.loop(0, SPAN // 16)
            def _(j):
                sl = pl.ds(j * 16, 16)
                sb[sl] = sb[sl] + ab[sl]

        @pl.loop(0, SPAN // 16)
        def _(j):
            sl = pl.ds(j * 16, 16)
            s = sb[sl]
            deg = jnp.zeros((16,), jnp.float32)
            for q in range(4):
                dq = jnp.sum(jnp.where(iota == 4 * q + 3, s, 0.0))
                deg = jnp.where((iota >= 4 * q) & (iota < 4 * q + 4), dq, deg)
            deg = jnp.maximum(deg, 1.0)
            xb[sl] = xb[sl] + jnp.where(lane < 3, s, 0.0) / deg

        pltpu.sync_copy(xb, xo_hbm.at[pl.ds(start, SPAN)])

    return k(xflat, xacc_p)


def _tc_edge(hs, hd, r2, wa, wb, wc, be1, we2, be2, wx1, bx1, wx2):
    BE = 512

    def body(hs_ref, hd_ref, r2_ref, wa_ref, wb_ref, wc_ref, be1_ref,
             we2_ref, be2_ref, wx1_ref, bx1_ref, wx2_ref, m_ref, w_ref):
        f32 = jnp.float32
        hi = lax.Precision.HIGHEST
        m1 = (jnp.dot(hd_ref[...], wa_ref[...], precision=hi,
                      preferred_element_type=f32)
              + jnp.dot(hs_ref[...], wb_ref[...], precision=hi,
                        preferred_element_type=f32)
              + r2_ref[...][:, None] * wc_ref[...]
              + be1_ref[...])
        m1 = _silu(m1)
        m = _silu(jnp.dot(m1, we2_ref[...], precision=hi,
                          preferred_element_type=f32) + be2_ref[...])
        t = _silu(jnp.dot(m, wx1_ref[...], precision=hi,
                          preferred_element_type=f32) + bx1_ref[...])
        m_ref[...] = m
        w_ref[...] = jnp.sum(t * wx2_ref[...], axis=1)

    rep = lambda s: pl.BlockSpec(s, lambda i: tuple(0 for _ in s))
    return pl.pallas_call(
        body,
        grid=(E // BE,),
        in_specs=[
            pl.BlockSpec((BE, H), lambda i: (i, 0)),
            pl.BlockSpec((BE, H), lambda i: (i, 0)),
            pl.BlockSpec((BE,), lambda i: (i,)),
            rep((H, H)), rep((H, H)), rep((1, H)), rep((1, H)),
            rep((H, H)), rep((1, H)), rep((H, H)), rep((1, H)), rep((1, H)),
        ],
        out_specs=[
            pl.BlockSpec((BE, H), lambda i: (i, 0)),
            pl.BlockSpec((BE,), lambda i: (i,)),
        ],
        out_shape=[
            jax.ShapeDtypeStruct((E, H), jnp.float32),
            jax.ShapeDtypeStruct((E,), jnp.float32),
        ],
    )(hs, hd, r2, wa, wb, wc, be1, we2, be2, wx1, bx1, wx2)


def _tc_node(h, p0, p1, wh1a, wh1b, bh1, wh2, bh2):
    BN = 1000

    def body(h_ref, p0_ref, p1_ref, a_ref, b_ref, b1_ref, w2_ref, b2_ref,
             o_ref):
        f32 = jnp.float32
        hi = lax.Precision.HIGHEST
        magg = p0_ref[...] + p1_ref[...]
        u = _silu(jnp.dot(h_ref[...], a_ref[...], precision=hi,
                          preferred_element_type=f32)
                  + jnp.dot(magg, b_ref[...], precision=hi,
                            preferred_element_type=f32) + b1_ref[...])
        o_ref[...] = h_ref[...] + _silu(
            jnp.dot(u, w2_ref[...], precision=hi,
                    preferred_element_type=f32) + b2_ref[...])

    rep = lambda s: pl.BlockSpec(s, lambda i: tuple(0 for _ in s))
    return pl.pallas_call(
        body,
        grid=(N // BN,),
        in_specs=[
            pl.BlockSpec((BN, H), lambda i: (i, 0)),
            pl.BlockSpec((BN, H), lambda i: (i, 0)),
            pl.BlockSpec((BN, H), lambda i: (i, 0)),
            rep((H, H)), rep((H, H)), rep((1, H)), rep((H, H)), rep((1, H)),
        ],
        out_specs=pl.BlockSpec((BN, H), lambda i: (i, 0)),
        out_shape=jax.ShapeDtypeStruct((N, H), jnp.float32),
    )(h, p0, p1, wh1a, wh1b, bh1, wh2, bh2)


def _tc_embed(x, w, b):
    BN = 1000
    n, fin = x.shape
    fout = w.shape[1]

    def body(x_ref, w_ref, b_ref, o_ref):
        o_ref[...] = jnp.dot(x_ref[...], w_ref[...],
                             precision=lax.Precision.HIGHEST,
                             preferred_element_type=jnp.float32) + b_ref[...]

    rep = lambda s: pl.BlockSpec(s, lambda i: tuple(0 for _ in s))
    return pl.pallas_call(
        body,
        grid=(n // BN,),
        in_specs=[pl.BlockSpec((BN, fin), lambda i: (i, 0)),
                  rep((fin, fout)), rep((1, fout))],
        out_specs=pl.BlockSpec((BN, fout), lambda i: (i, 0)),
        out_shape=jax.ShapeDtypeStruct((n, fout), jnp.float32),
    )(x, w, b)


def kernel(edge_index, feat, coordinate, params):
    depth = params["We1"].shape[0]
    src = edge_index[0]
    dst = edge_index[1]
    xflat = jnp.pad(coordinate, ((0, 0), (0, XP - 3))).reshape(XF)
    zrows = jnp.zeros((ZRW, H), jnp.float32)
    z1d = jnp.zeros((XF,), jnp.float32)

    h = _tc_embed(feat, params["Win"], params["bin"].reshape(1, H))

    for i in range(depth):
        we1 = params["We1"][i]
        hs, hd, diff, r2 = _sc_gather(h, xflat, src, dst)
        m, w = _tc_edge(
            hs, hd, r2,
            we1[0:H], we1[H:2 * H], we1[2 * H:2 * H + 1],
            params["be1"][i].reshape(1, H),
            params["We2"][i], params["be2"][i].reshape(1, H),
            params["Wx1"][i], params["bx1"][i].reshape(1, H),
            params["Wx2"][i].reshape(1, H),
        )
        magg_p = _sc_scatter_m(m, dst, zrows)
        xacc_p = _sc_scatter_x(diff, w, dst, z1d)
        xflat = _sc_xupdate(xflat, xacc_p)
        wh1 = params["Wh1"][i]
        h = _tc_node(h, magg_p[0], magg_p[1],
                     wh1[0:H], wh1[H:2 * H],
                     params["bh1"][i].reshape(1, H),
                     params["Wh2"][i], params["bh2"][i].reshape(1, H))

    out = _tc_embed(h, params["Wout"], params["bout"].reshape(1, H))
    return (out, xflat.reshape(N, XP)[:, :3])
